# Initial kernel scaffold; baseline (speedup 1.0000x reference)
import jax
import jax.numpy as jnp
from jax.experimental import pallas as pl


def kernel(means3d, opacities, covariances, grid_coords, vol_range):
    raise NotImplementedError("write your pallas kernel here")


# speedup vs baseline: 4.9135x; 4.9135x over previous
import jax
import jax.numpy as jnp
from jax.experimental import pallas as pl
from jax.experimental.pallas import tpu as pltpu


def _prep_kernel(vr_ref, p_ref, c_ref):
    mx = p_ref[0:1, :]
    my = p_ref[1:2, :]
    mz = p_ref[2:3, :]
    a = p_ref[3:4, :]
    b = p_ref[4:5, :]
    c = p_ref[5:6, :]
    d = p_ref[6:7, :]
    e = p_ref[7:8, :]
    f = p_ref[8:9, :]
    op = p_ref[9:10, :]

    m00 = b * c - f * f
    m01 = e * f - d * c
    m02 = d * f - b * e
    det = a * m00 + d * m01 + e * m02
    rdet = 1.0 / det
    ixx = m00 * rdet
    iyy = (a * c - e * e) * rdet
    izz = (a * b - d * d) * rdet
    ixy = m01 * rdet
    ixz = m02 * rdet
    iyz = (d * e - a * f) * rdet

    sx = jnp.sqrt(a)
    sy = jnp.sqrt(b)
    sz = jnp.sqrt(c)
    lo0, lo1, lo2 = vr_ref[0], vr_ref[1], vr_ref[2]
    hi0, hi1, hi2 = vr_ref[3], vr_ref[4], vr_ref[5]
    mask = ((mx + 3.0 * sx > lo0) & (my + 3.0 * sy > lo1)
            & (mz + 3.0 * sz > lo2) & (mx - 3.0 * sx < hi0)
            & (my - 3.0 * sy < hi1) & (mz - 3.0 * sz < hi2))
    w = jnp.where(mask, op, 0.0)

    c_ref[0:1, :] = mx
    c_ref[1:2, :] = my
    c_ref[2:3, :] = mz
    c_ref[3:4, :] = -0.5 * ixx
    c_ref[4:5, :] = -0.5 * iyy
    c_ref[5:6, :] = -0.5 * izz
    c_ref[6:7, :] = -ixy
    c_ref[7:8, :] = -ixz
    c_ref[8:9, :] = -iyz
    c_ref[9:10, :] = w
    c_ref[10:16, :] = jnp.zeros_like(c_ref[10:16, :])


def _main_kernel(c_ref, xyz_ref, o_ref):
    x = xyz_ref[0:1, :]
    y = xyz_ref[1:2, :]
    z = xyz_ref[2:3, :]
    mx = c_ref[:, 0:1]
    my = c_ref[:, 1:2]
    mz = c_ref[:, 2:3]
    qxx = c_ref[:, 3:4]
    qyy = c_ref[:, 4:5]
    qzz = c_ref[:, 5:6]
    qxy = c_ref[:, 6:7]
    qxz = c_ref[:, 7:8]
    qyz = c_ref[:, 8:9]
    w = c_ref[:, 9:10]

    dx = x - mx
    dy = y - my
    dz = z - mz
    expo = (dx * (qxx * dx + qxy * dy + qxz * dz)
            + dy * (qyy * dy + qyz * dz)
            + qzz * dz * dz)
    dens = w * jnp.exp(expo)
    o_ref[...] = jnp.sum(dens, axis=0, keepdims=True)


def kernel(means3d, opacities, covariances, grid_coords, vol_range,
           interpret=False):
    G = means3d.shape[0]
    grid_shape = grid_coords.shape[:-1]
    N = grid_shape[0] * grid_shape[1] * grid_shape[2]
    VB = 640
    assert N % VB == 0
    nblocks = N // VB

    f32 = jnp.float32
    covrows = jnp.stack([
        covariances[:, 0, 0], covariances[:, 1, 1], covariances[:, 2, 2],
        covariances[:, 0, 1], covariances[:, 0, 2], covariances[:, 1, 2],
    ], axis=0)
    p = jnp.concatenate([
        means3d.T.astype(f32), covrows.astype(f32),
        opacities[None, :].astype(f32), jnp.zeros((6, G), f32),
    ], axis=0)

    coeff_t = pl.pallas_call(
        _prep_kernel,
        out_shape=jax.ShapeDtypeStruct((16, G), f32),
        in_specs=[
            pl.BlockSpec(memory_space=pltpu.SMEM),
            pl.BlockSpec((16, G), lambda: (0, 0)),
        ],
        out_specs=pl.BlockSpec((16, G), lambda: (0, 0)),
        name="gv_prep",
        interpret=interpret,
    )(vol_range.astype(f32), p)

    coeff = coeff_t.T
    xyz = grid_coords.reshape(-1, 3).T.astype(f32)

    out = pl.pallas_call(
        _main_kernel,
        out_shape=jax.ShapeDtypeStruct((1, N), f32),
        grid=(nblocks,),
        in_specs=[
            pl.BlockSpec((G, 16), lambda i: (0, 0)),
            pl.BlockSpec((3, VB), lambda i: (0, i)),
        ],
        out_specs=pl.BlockSpec((1, VB), lambda i: (0, i)),
        compiler_params=pltpu.CompilerParams(
            dimension_semantics=("parallel",),
            vmem_limit_bytes=48 * 1024 * 1024,
        ),
        name="gv_main",
        interpret=interpret,
    )(coeff, xyz)

    return out.reshape(*grid_shape, 1)


# speedup vs baseline: 6.4019x; 1.3029x over previous
import jax
import jax.numpy as jnp
from jax.experimental import pallas as pl
from jax.experimental.pallas import tpu as pltpu


def _prep_kernel(vr_ref, p_ref, c_ref):
    mx = p_ref[0:1, :]
    my = p_ref[1:2, :]
    mz = p_ref[2:3, :]
    a = p_ref[3:4, :]
    b = p_ref[4:5, :]
    c = p_ref[5:6, :]
    d = p_ref[6:7, :]
    e = p_ref[7:8, :]
    f = p_ref[8:9, :]
    op = p_ref[9:10, :]

    m00 = b * c - f * f
    m01 = e * f - d * c
    m02 = d * f - b * e
    det = a * m00 + d * m01 + e * m02
    rdet = 1.0 / det
    qxx = -0.5 * m00 * rdet
    qyy = -0.5 * (a * c - e * e) * rdet
    qzz = -0.5 * (a * b - d * d) * rdet
    qxy = -m01 * rdet
    qxz = -m02 * rdet
    qyz = -(d * e - a * f) * rdet

    sx = jnp.sqrt(a)
    sy = jnp.sqrt(b)
    sz = jnp.sqrt(c)
    lo0, lo1, lo2 = vr_ref[0], vr_ref[1], vr_ref[2]
    hi0, hi1, hi2 = vr_ref[3], vr_ref[4], vr_ref[5]
    mask = ((mx + 3.0 * sx > lo0) & (my + 3.0 * sy > lo1)
            & (mz + 3.0 * sz > lo2) & (mx - 3.0 * sx < hi0)
            & (my - 3.0 * sy < hi1) & (mz - 3.0 * sz < hi2))
    logw = jnp.where(mask & (op > 0.0), jnp.log(jnp.maximum(op, 1e-30)),
                     -1e30)

    lx = -2.0 * qxx * mx - qxy * my - qxz * mz
    ly = -2.0 * qyy * my - qxy * mx - qyz * mz
    lz = -2.0 * qzz * mz - qxz * mx - qyz * my
    c0 = (qxx * mx * mx + qyy * my * my + qzz * mz * mz
          + qxy * mx * my + qxz * mx * mz + qyz * my * mz + logw)

    c_ref[0:1, :] = qxx
    c_ref[1:2, :] = qyy
    c_ref[2:3, :] = qzz
    c_ref[3:4, :] = qxy
    c_ref[4:5, :] = qxz
    c_ref[5:6, :] = qyz
    c_ref[6:7, :] = lx
    c_ref[7:8, :] = ly
    c_ref[8:9, :] = lz
    c_ref[9:10, :] = c0
    c_ref[10:16, :] = jnp.zeros_like(c_ref[10:16, :])


def _main_kernel(c_ref, xyz_ref, o_ref):
    x = xyz_ref[0:1, :]
    y = xyz_ref[1:2, :]
    z = xyz_ref[2:3, :]
    vb = x.shape[1]
    phi = jnp.concatenate([
        x * x, y * y, z * z, x * y, x * z, y * z, x, y, z,
        jnp.ones_like(x), jnp.zeros((6, vb), jnp.float32),
    ], axis=0)
    expo = jnp.dot(c_ref[...], phi, preferred_element_type=jnp.float32,
                   precision=jax.lax.Precision.HIGHEST)
    o_ref[...] = jnp.sum(jnp.exp(expo), axis=0, keepdims=True)


def kernel(means3d, opacities, covariances, grid_coords, vol_range,
           interpret=False):
    G = means3d.shape[0]
    grid_shape = grid_coords.shape[:-1]
    N = grid_shape[0] * grid_shape[1] * grid_shape[2]
    VB = 640
    assert N % VB == 0
    nblocks = N // VB

    f32 = jnp.float32
    covrows = jnp.stack([
        covariances[:, 0, 0], covariances[:, 1, 1], covariances[:, 2, 2],
        covariances[:, 0, 1], covariances[:, 0, 2], covariances[:, 1, 2],
    ], axis=0)
    p = jnp.concatenate([
        means3d.T.astype(f32), covrows.astype(f32),
        opacities[None, :].astype(f32), jnp.zeros((6, G), f32),
    ], axis=0)

    coeff_t = pl.pallas_call(
        _prep_kernel,
        out_shape=jax.ShapeDtypeStruct((16, G), f32),
        in_specs=[
            pl.BlockSpec(memory_space=pltpu.SMEM),
            pl.BlockSpec((16, G), lambda: (0, 0)),
        ],
        out_specs=pl.BlockSpec((16, G), lambda: (0, 0)),
        name="gv_prep",
        interpret=interpret,
    )(vol_range.astype(f32), p)

    coeff = coeff_t.T
    xyz = grid_coords.reshape(-1, 3).T.astype(f32)

    out = pl.pallas_call(
        _main_kernel,
        out_shape=jax.ShapeDtypeStruct((1, N), f32),
        grid=(nblocks,),
        in_specs=[
            pl.BlockSpec((G, 16), lambda i: (0, 0)),
            pl.BlockSpec((3, VB), lambda i: (0, i)),
        ],
        out_specs=pl.BlockSpec((1, VB), lambda i: (0, i)),
        compiler_params=pltpu.CompilerParams(
            dimension_semantics=("parallel",),
            vmem_limit_bytes=48 * 1024 * 1024,
        ),
        name="gv_main",
        interpret=interpret,
    )(coeff, xyz)

    return out.reshape(*grid_shape, 1)
